# baseline (speedup 1.0000x reference)
import functools

import jax
import jax.numpy as jnp
import numpy as np
from jax import lax
from jax.experimental import pallas as pl
from jax.experimental.pallas import tpu as pltpu
from jax.experimental.pallas import tpu_sc as plsc

N = 2048
D = 128
NG = 64
NL = 3
NT = 100
CUT = 5.0
CELL = 30.0

NC = 2
NS = 16
NW = NC * NS
CAP = 2048
E = NW * CAP
CHUNK = 128
NCHMAX = CAP // CHUNK
ROWS_PER_TILE = N // NW

LOG2 = float(np.log(2.0))
F32 = jnp.float32


def _ssp(x):
    return jnp.maximum(x, 0.0) + jnp.log(1.0 + jnp.exp(-jnp.abs(x))) - LOG2


_BR = 256


def _dist_body(xi_ref, xjt_ref, out_ref):
    p = pl.program_id(0)
    xi = xi_ref[...]
    xjt = xjt_ref[...]
    dsq = jnp.zeros((_BR, N), F32)
    for k in range(3):
        dk = xjt[k:k + 1, :] - xi[:, k:k + 1]
        off = -(dk >= 0.5 * CELL).astype(F32) + (dk < -0.5 * CELL).astype(F32)
        dk = dk + off * CELL
        dsq = dsq + dk * dk
    rows = p * _BR + lax.broadcasted_iota(jnp.int32, (_BR, N), 0)
    cols = lax.broadcasted_iota(jnp.int32, (_BR, N), 1)
    mask = (cols > rows) & (dsq < CUT * CUT) & (dsq != 0.0)
    out_ref[...] = jnp.where(mask, jnp.sqrt(dsq + 1e-12), -1.0)


def _distmat(xyz, xyzt):
    return pl.pallas_call(
        _dist_body,
        grid=(N // _BR,),
        in_specs=[
            pl.BlockSpec((_BR, 3), lambda p: (p, 0)),
            pl.BlockSpec((3, N), lambda p: (0, 0)),
        ],
        out_specs=pl.BlockSpec((_BR, N), lambda p: (p, 0)),
        out_shape=jax.ShapeDtypeStruct((N, N), F32),
    )(xyz, xyzt)


NRP = ROWS_PER_TILE + 1


@functools.partial(
    pl.kernel,
    out_type=[
        jax.ShapeDtypeStruct((NW, CAP), jnp.int32),
        jax.ShapeDtypeStruct((NW, CAP), F32),
        jax.ShapeDtypeStruct((NW, 16), jnp.int32),
        jax.ShapeDtypeStruct((NW, NRP * 16), jnp.int32),
    ],
    mesh=plsc.VectorSubcoreMesh(core_axis_name="c", subcore_axis_name="s",
                                num_cores=NC, num_subcores=NS),
    compiler_params=pltpu.CompilerParams(
        needs_layout_passes=False, use_tc_tiling_on_sc=False),
    scratch_types=[
        pltpu.VMEM((N,), F32),
        pltpu.VMEM((CAP + 16,), jnp.int32),
        pltpu.VMEM((CAP + 16,), F32),
        pltpu.VMEM((16,), jnp.int32),
        pltpu.VMEM((NRP * 16,), jnp.int32),
    ],
)
def _compact(dm, eidx_o, edist_o, cnt_o, rpt_o,
             row_v, eidx_v, edist_v, cnt_v, rpt_v):
    cid = lax.axis_index("c")
    sid = lax.axis_index("s")
    wid = sid * NC + cid

    def init_body(k, _):
        eidx_v[pl.ds(k * 16, 16)] = jnp.zeros((16,), jnp.int32)
        edist_v[pl.ds(k * 16, 16)] = jnp.full((16,), -1.0, F32)
        return 0

    lax.fori_loop(0, (CAP + 16) // 16, init_body, 0)
    rpt_v[pl.ds(0, 16)] = jnp.zeros((16,), jnp.int32)

    def row_body(rr, p):
        r = wid + rr * NW
        pltpu.sync_copy(dm.at[r], row_v)
        cstart = (r + 1) >> 4

        def chunk_body(c, p):
            v = row_v[pl.ds(c * 16, 16)]
            m = v >= 0.0
            cnt = jnp.sum(m.astype(jnp.int32))
            idx = r * N + c * 16 + lax.iota(jnp.int32, 16)
            pc = jnp.minimum(p, CAP)

            @pl.when(cnt > 0)
            def _():
                plsc.store_compressed(eidx_v.at[pl.ds(pc, 16)], idx, mask=m)
                plsc.store_compressed(edist_v.at[pl.ds(pc, 16)], v, mask=m)

            return p + cnt

        p = lax.fori_loop(cstart, N // 16, chunk_body, p)
        rpt_v[pl.ds((rr + 1) * 16, 16)] = (
            jnp.zeros((16,), jnp.int32) + jnp.minimum(p, CAP))
        return p

    p = lax.fori_loop(0, ROWS_PER_TILE, row_body, jnp.int32(0))
    cnt_v[...] = jnp.zeros((16,), jnp.int32) + jnp.minimum(p, CAP)
    pltpu.sync_copy(eidx_v.at[pl.ds(0, CAP)], eidx_o.at[wid])
    pltpu.sync_copy(edist_v.at[pl.ds(0, CAP)], edist_o.at[wid])
    pltpu.sync_copy(cnt_v, cnt_o.at[wid])
    pltpu.sync_copy(rpt_v, rpt_o.at[wid])


_BE = 4096


def _edge_body(d_ref, wf1_ref, bf1_ref, wf2_ref, bf2_ref, o0, o1, o2):
    d = d_ref[...]
    width = CUT / (NG - 1)
    cent = lax.broadcasted_iota(jnp.int32, (1, NG), 1).astype(F32) * width
    e = jnp.exp(-0.5 * ((d - cent) / width) ** 2)
    fcut = 0.5 * (jnp.cos(np.pi / CUT * d) + 1.0)
    valid = (d >= 0.0).astype(F32)
    scale = fcut * valid
    for l, o in enumerate((o0, o1, o2)):
        t = _ssp(jnp.dot(e, wf1_ref[l], preferred_element_type=F32)
                 + bf1_ref[l])
        t = jnp.dot(t, wf2_ref[l], preferred_element_type=F32) + bf2_ref[l]
        o[...] = t * scale


def _edge_mlp(edist, wf1s, bf1s, wf2s, bf2s):
    return pl.pallas_call(
        _edge_body,
        grid=(E // _BE,),
        in_specs=[
            pl.BlockSpec((_BE, 1), lambda p: (p, 0)),
            pl.BlockSpec((NL, NG, D), lambda p: (0, 0, 0)),
            pl.BlockSpec((NL, 1, D), lambda p: (0, 0, 0)),
            pl.BlockSpec((NL, D, D), lambda p: (0, 0, 0)),
            pl.BlockSpec((NL, 1, D), lambda p: (0, 0, 0)),
        ],
        out_specs=[pl.BlockSpec((_BE, D), lambda p: (p, 0))] * NL,
        out_shape=[jax.ShapeDtypeStruct((E, D), F32)] * NL,
    )(edist, wf1s, bf1s, wf2s, bf2s)


def _embed_body(z_ref, emb_ref, win_ref, x_ref, h_ref):
    z = z_ref[...]
    oh = (z == lax.broadcasted_iota(jnp.int32, (1, NT), 1)).astype(F32)
    x = jnp.dot(oh, emb_ref[...], preferred_element_type=F32)
    x_ref[...] = x
    h_ref[...] = jnp.dot(x, win_ref[...], preferred_element_type=F32)


def _embed_h(z2, emb, win):
    return pl.pallas_call(
        _embed_body,
        out_shape=[jax.ShapeDtypeStruct((N, D), F32)] * 2,
    )(z2, emb, win)


@functools.partial(
    pl.kernel,
    out_type=[
        jax.ShapeDtypeStruct((NC, N, D), F32),
        jax.ShapeDtypeStruct((NW, ROWS_PER_TILE, D), F32),
    ],
    mesh=plsc.VectorSubcoreMesh(core_axis_name="c", subcore_axis_name="s",
                                num_cores=NC, num_subcores=NS),
    compiler_params=pltpu.CompilerParams(
        needs_layout_passes=False, use_tc_tiling_on_sc=False),
    scratch_types=[
        pltpu.VMEM((CAP,), jnp.int32),
        pltpu.VMEM((NCHMAX, CHUNK), jnp.int32),
        pltpu.VMEM((NCHMAX, CHUNK), jnp.int32),
        pltpu.VMEM((16,), jnp.int32),
        pltpu.VMEM((NRP * 16,), jnp.int32),
        pltpu.VMEM((CHUNK, D), F32),
        pltpu.VMEM((CHUNK, D), F32),
        pltpu.VMEM((CHUNK, D), F32),
        pltpu.VMEM((CHUNK, D), F32),
        pltpu.VMEM((CHUNK, D), F32),
        pltpu.VMEM((ROWS_PER_TILE, D), F32),
        pltpu.VMEM_SHARED((N, D), F32),
        pltpu.SemaphoreType.DMA,
        pltpu.SemaphoreType.DMA,
        pltpu.SemaphoreType.DMA,
        pltpu.SemaphoreType.DMA,
        pltpu.SemaphoreType.DMA,
        pltpu.SemaphoreType.DMA,
    ],
)
def _scatter(h, w3, eidx, cnts, rpt, agg_o, iagg_o,
             eflat_v, iidx_v, jidx_v, cnt_v, rpt_v,
             hj0_v, hi0_v, hj1_v, hi1_v, w_v, iacc_v, agg_s,
             gj0, gi0, gj1, gi1, s0j, s1j):
    cid = lax.axis_index("c")
    sid = lax.axis_index("s")
    wid = sid * NC + cid
    hjs = (hj0_v, hj1_v)
    his = (hi0_v, hi1_v)
    gsems = ((gj0, gi0), (gj1, gi1))
    ssems = (s0j, s1j)

    pltpu.sync_copy(eidx.at[wid], eflat_v)
    pltpu.sync_copy(cnts.at[wid], cnt_v)
    pltpu.sync_copy(rpt.at[wid], rpt_v)

    def split_body(c, _):
        for t in range(CHUNK // 16):
            v = eflat_v[pl.ds(c * CHUNK + t * 16, 16)]
            iidx_v[c, pl.ds(t * 16, 16)] = v >> 11
            jidx_v[c, pl.ds(t * 16, 16)] = v & (N - 1)
        return 0

    lax.fori_loop(0, NCHMAX, split_body, 0)

    def izero_body(r, _):
        for t in range(D // 16):
            iacc_v[r, pl.ds(t * 16, 16)] = jnp.zeros((16,), F32)
        return 0

    lax.fori_loop(0, ROWS_PER_TILE, izero_body, 0)

    def zero_body(r, _):
        for t in range(D // 16):
            hj0_v[r, pl.ds(t * 16, 16)] = jnp.zeros((16,), F32)
        return 0

    lax.fori_loop(0, CHUNK, zero_body, 0)
    pltpu.sync_copy(hj0_v, agg_s.at[pl.ds(sid * CHUNK, CHUNK)])
    plsc.subcore_barrier()

    nch = jnp.max((cnt_v[...] + (CHUNK - 1)) >> 7)

    def start_gathers(c, s):
        pltpu.make_async_copy(h.at[jidx_v.at[c]], hjs[s], gsems[s][0]).start()
        pltpu.make_async_copy(h.at[iidx_v.at[c]], his[s], gsems[s][1]).start()

    def wait_gathers(c, s):
        pltpu.make_async_copy(h.at[jidx_v.at[c]], hjs[s], gsems[s][0]).wait()
        pltpu.make_async_copy(h.at[iidx_v.at[c]], his[s], gsems[s][1]).wait()

    def start_scatter(c, s):
        pltpu.async_copy(his[s], agg_s.at[jidx_v.at[c]], ssems[s], add=True)

    def wait_scatter(c, s):
        pltpu.make_async_copy(his[s], agg_s.at[jidx_v.at[c]],
                              ssems[s]).wait()

    @pl.when(nch > 0)
    def _():
        start_gathers(0, 0)

    def process(c, s):
        wait_gathers(c, s)
        pltpu.sync_copy(w3.at[wid, pl.ds(c * CHUNK, CHUNK)], w_v)
        hj_v = hjs[s]
        hi_v = his[s]

        def mul_body(r, _):
            for t in range(D // 16):
                sl = pl.ds(t * 16, 16)
                w = w_v[r, sl]
                hj_v[r, sl] = hj_v[r, sl] * w
                hi_v[r, sl] = hi_v[r, sl] * w
            return 0

        lax.fori_loop(0, CHUNK, mul_body, 0)
        start_scatter(c, s)

        @pl.when(c + 1 < nch)
        def _():
            @pl.when(c >= 1)
            def _():
                wait_scatter(c - 1, 1 - s)

            start_gathers(c + 1, 1 - s)

        base = c * CHUNK

        def irow_body(rr, _):
            lo = jnp.max(rpt_v[pl.ds(rr * 16, 16)])
            hi_ = jnp.max(rpt_v[pl.ds((rr + 1) * 16, 16)])
            lo = jnp.maximum(lo, base)
            hi_ = jnp.minimum(hi_, base + CHUNK)

            @pl.when(hi_ > lo)
            def _():
                def acc_body(k, _):
                    r = k - base
                    for t in range(D // 16):
                        sl = pl.ds(t * 16, 16)
                        iacc_v[rr, sl] = iacc_v[rr, sl] + hj_v[r, sl]
                    return 0

                lax.fori_loop(lo, hi_, acc_body, 0)

            return 0

        lax.fori_loop(0, ROWS_PER_TILE, irow_body, 0)

    def chunk_body(c, _):
        @pl.when((c & 1) == 0)
        def _():
            process(c, 0)

        @pl.when((c & 1) == 1)
        def _():
            process(c, 1)

        return 0

    lax.fori_loop(0, nch, chunk_body, 0)

    nch_even = (nch & 1) == 0

    @pl.when((nch >= 2) & nch_even)
    def _():
        wait_scatter(nch - 2, 0)

    @pl.when((nch >= 2) & ~nch_even)
    def _():
        wait_scatter(nch - 2, 1)

    @pl.when((nch >= 1) & ~nch_even)
    def _():
        wait_scatter(nch - 1, 0)

    @pl.when((nch >= 1) & nch_even)
    def _():
        wait_scatter(nch - 1, 1)

    pltpu.sync_copy(iacc_v, iagg_o.at[wid])
    plsc.subcore_barrier()
    pltpu.sync_copy(agg_s.at[pl.ds(sid * CHUNK, CHUNK)],
                    agg_o.at[cid, pl.ds(sid * CHUNK, CHUNK)])


def _node_body(aggs_ref, ip_ref, x_ref, wd1_ref, bd1_ref, wd2_ref, bd2_ref,
               win_ref, xo_ref, ho_ref):
    agg = aggs_ref[0] + aggs_ref[1] + ip_ref[...]
    t = _ssp(jnp.dot(agg, wd1_ref[...], preferred_element_type=F32)
             + bd1_ref[...])
    x = x_ref[...] + jnp.dot(t, wd2_ref[...], preferred_element_type=F32) \
        + bd2_ref[...]
    xo_ref[...] = x
    ho_ref[...] = jnp.dot(x, win_ref[...], preferred_element_type=F32)


def _node(aggs, ipart, x, wd1, bd1, wd2, bd2, win):
    return pl.pallas_call(
        _node_body,
        out_shape=[jax.ShapeDtypeStruct((N, D), F32)] * 2,
    )(aggs, ipart, x, wd1, bd1, wd2, bd2, win)


def _final_body(aggs_ref, ip_ref, x_ref, wd1_ref, bd1_ref, wd2_ref, bd2_ref,
                wr1_ref, br1_ref, wr2_ref, br2_ref, out_ref):
    agg = aggs_ref[0] + aggs_ref[1] + ip_ref[...]
    t = _ssp(jnp.dot(agg, wd1_ref[...], preferred_element_type=F32)
             + bd1_ref[...])
    x = x_ref[...] + jnp.dot(t, wd2_ref[...], preferred_element_type=F32) \
        + bd2_ref[...]
    ea = jnp.dot(_ssp(jnp.dot(x, wr1_ref[...], preferred_element_type=F32)
                      + br1_ref[...]),
                 wr2_ref[...], preferred_element_type=F32) + br2_ref[...]
    out_ref[...] = jnp.sum(ea).reshape(1, 1)


def _final(aggs, ipart, x, wd1, bd1, wd2, bd2, wr1, br1, wr2, br2):
    return pl.pallas_call(
        _final_body,
        out_shape=jax.ShapeDtypeStruct((1, 1), F32),
    )(aggs, ipart, x, wd1, bd1, wd2, bd2, wr1, br1, wr2, br2)


def kernel(xyz, params, z):
    xyz = xyz.astype(F32)
    dm = _distmat(xyz, xyz.T)
    eidx, edist, cnts, rpt = _compact(dm)

    wf1s = jnp.stack([params['Wf1_%d' % l] for l in range(NL)])
    bf1s = jnp.stack([params['bf1_%d' % l].reshape(1, D) for l in range(NL)])
    wf2s = jnp.stack([params['Wf2_%d' % l] for l in range(NL)])
    bf2s = jnp.stack([params['bf2_%d' % l].reshape(1, D) for l in range(NL)])
    ws = _edge_mlp(edist.reshape(E, 1), wf1s, bf1s, wf2s, bf2s)

    x, h = _embed_h(z.astype(jnp.int32).reshape(N, 1), params['embed'],
                    params['Win_0'])
    out = None
    for l in range(NL):
        w3 = ws[l].reshape(NW, CAP, D)
        aggs, iagg = _scatter(h, w3, eidx, cnts, rpt)
        ipart = jnp.transpose(iagg, (1, 0, 2)).reshape(N, D)
        wd1 = params['Wd1_%d' % l]
        bd1 = params['bd1_%d' % l].reshape(1, D)
        wd2 = params['Wd2_%d' % l]
        bd2 = params['bd2_%d' % l].reshape(1, D)
        if l < NL - 1:
            x, h = _node(aggs, ipart, x, wd1, bd1, wd2, bd2,
                         params['Win_%d' % (l + 1)])
        else:
            out = _final(aggs, ipart, x, wd1, bd1, wd2, bd2,
                         params['Wr1'], params['br1'].reshape(1, 64),
                         params['Wr2'], params['br2'].reshape(1, 1))
    return out[0, 0]


# speedup vs baseline: 1.2109x; 1.2109x over previous
import functools

import jax
import jax.numpy as jnp
import numpy as np
from jax import lax
from jax.experimental import pallas as pl
from jax.experimental.pallas import tpu as pltpu
from jax.experimental.pallas import tpu_sc as plsc

N = 2048
D = 128
NG = 64
NL = 3
NT = 100
CUT = 5.0
CELL = 30.0

NC = 2
NS = 16
NW = NC * NS
CAP = 2048
E = NW * CAP
CHUNK = 128
NCHMAX = CAP // CHUNK
ROWS_PER_TILE = N // NW

LOG2 = float(np.log(2.0))
F32 = jnp.float32


def _ssp(x):
    return jnp.maximum(x, 0.0) + jnp.log(1.0 + jnp.exp(-jnp.abs(x))) - LOG2


_BR = 256


def _dist_body(xi_ref, xjt_ref, out_ref):
    p = pl.program_id(0)
    xi = xi_ref[...]
    xjt = xjt_ref[...]
    dsq = jnp.zeros((_BR, N), F32)
    for k in range(3):
        dk = xjt[k:k + 1, :] - xi[:, k:k + 1]
        off = -(dk >= 0.5 * CELL).astype(F32) + (dk < -0.5 * CELL).astype(F32)
        dk = dk + off * CELL
        dsq = dsq + dk * dk
    rows = p * _BR + lax.broadcasted_iota(jnp.int32, (_BR, N), 0)
    cols = lax.broadcasted_iota(jnp.int32, (_BR, N), 1)
    mask = (cols > rows) & (dsq < CUT * CUT) & (dsq != 0.0)
    out_ref[...] = jnp.where(mask, jnp.sqrt(dsq + 1e-12), -1.0)


def _distmat(xyz, xyzt):
    return pl.pallas_call(
        _dist_body,
        grid=(N // _BR,),
        in_specs=[
            pl.BlockSpec((_BR, 3), lambda p: (p, 0)),
            pl.BlockSpec((3, N), lambda p: (0, 0)),
        ],
        out_specs=pl.BlockSpec((_BR, N), lambda p: (p, 0)),
        out_shape=jax.ShapeDtypeStruct((N, N), F32),
    )(xyz, xyzt)


NRP = ROWS_PER_TILE + 1


@functools.partial(
    pl.kernel,
    out_type=[
        jax.ShapeDtypeStruct((NW, CAP), jnp.int32),
        jax.ShapeDtypeStruct((NW, CAP), F32),
        jax.ShapeDtypeStruct((NW, 16), jnp.int32),
        jax.ShapeDtypeStruct((NW, NRP * 16), jnp.int32),
    ],
    mesh=plsc.VectorSubcoreMesh(core_axis_name="c", subcore_axis_name="s",
                                num_cores=NC, num_subcores=NS),
    compiler_params=pltpu.CompilerParams(
        needs_layout_passes=False, use_tc_tiling_on_sc=False),
    scratch_types=[
        pltpu.VMEM((N,), F32),
        pltpu.VMEM((CAP + 16,), jnp.int32),
        pltpu.VMEM((CAP + 16,), F32),
        pltpu.VMEM((16,), jnp.int32),
        pltpu.VMEM((NRP * 16,), jnp.int32),
    ],
)
def _compact(dm, eidx_o, edist_o, cnt_o, rpt_o,
             row_v, eidx_v, edist_v, cnt_v, rpt_v):
    cid = lax.axis_index("c")
    sid = lax.axis_index("s")
    wid = sid * NC + cid

    def init_body(k, _):
        eidx_v[pl.ds(k * 16, 16)] = jnp.zeros((16,), jnp.int32)
        edist_v[pl.ds(k * 16, 16)] = jnp.full((16,), -1.0, F32)
        return 0

    lax.fori_loop(0, (CAP + 16) // 16, init_body, 0)
    rpt_v[pl.ds(0, 16)] = jnp.zeros((16,), jnp.int32)

    def row_body(rr, p):
        r = wid + rr * NW
        pltpu.sync_copy(dm.at[r], row_v)
        cstart = (r + 1) >> 4

        def chunk_body(c, p):
            v = row_v[pl.ds(c * 16, 16)]
            m = v >= 0.0
            cnt = jnp.sum(m.astype(jnp.int32))
            idx = r * N + c * 16 + lax.iota(jnp.int32, 16)
            pc = jnp.minimum(p, CAP)

            @pl.when(cnt > 0)
            def _():
                plsc.store_compressed(eidx_v.at[pl.ds(pc, 16)], idx, mask=m)
                plsc.store_compressed(edist_v.at[pl.ds(pc, 16)], v, mask=m)

            return p + cnt

        p = lax.fori_loop(cstart, N // 16, chunk_body, p)
        rpt_v[pl.ds((rr + 1) * 16, 16)] = (
            jnp.zeros((16,), jnp.int32) + jnp.minimum(p, CAP))
        return p

    p = lax.fori_loop(0, ROWS_PER_TILE, row_body, jnp.int32(0))
    cnt_v[...] = jnp.zeros((16,), jnp.int32) + jnp.minimum(p, CAP)
    pltpu.sync_copy(eidx_v.at[pl.ds(0, CAP)], eidx_o.at[wid])
    pltpu.sync_copy(edist_v.at[pl.ds(0, CAP)], edist_o.at[wid])
    pltpu.sync_copy(cnt_v, cnt_o.at[wid])
    pltpu.sync_copy(rpt_v, rpt_o.at[wid])


_BE = 4096


def _edge_body(d_ref, wf1_ref, bf1_ref, wf2_ref, bf2_ref, o0, o1, o2):
    d = d_ref[...]
    width = CUT / (NG - 1)
    cent = lax.broadcasted_iota(jnp.int32, (1, NG), 1).astype(F32) * width
    e = jnp.exp(-0.5 * ((d - cent) / width) ** 2)
    fcut = 0.5 * (jnp.cos(np.pi / CUT * d) + 1.0)
    valid = (d >= 0.0).astype(F32)
    scale = fcut * valid
    for l, o in enumerate((o0, o1, o2)):
        t = _ssp(jnp.dot(e, wf1_ref[l], preferred_element_type=F32)
                 + bf1_ref[l])
        t = jnp.dot(t, wf2_ref[l], preferred_element_type=F32) + bf2_ref[l]
        o[...] = t * scale


def _edge_mlp(edist, wf1s, bf1s, wf2s, bf2s):
    return pl.pallas_call(
        _edge_body,
        grid=(E // _BE,),
        in_specs=[
            pl.BlockSpec((_BE, 1), lambda p: (p, 0)),
            pl.BlockSpec((NL, NG, D), lambda p: (0, 0, 0)),
            pl.BlockSpec((NL, 1, D), lambda p: (0, 0, 0)),
            pl.BlockSpec((NL, D, D), lambda p: (0, 0, 0)),
            pl.BlockSpec((NL, 1, D), lambda p: (0, 0, 0)),
        ],
        out_specs=[pl.BlockSpec((_BE, D), lambda p: (p, 0))] * NL,
        out_shape=[jax.ShapeDtypeStruct((E, D), F32)] * NL,
    )(edist, wf1s, bf1s, wf2s, bf2s)


def _embed_body(z_ref, emb_ref, win_ref, x_ref, h_ref):
    z = z_ref[...]
    oh = (z == lax.broadcasted_iota(jnp.int32, (1, NT), 1)).astype(F32)
    x = jnp.dot(oh, emb_ref[...], preferred_element_type=F32)
    x_ref[...] = x
    h_ref[...] = jnp.dot(x, win_ref[...], preferred_element_type=F32)


def _embed_h(z2, emb, win):
    return pl.pallas_call(
        _embed_body,
        out_shape=[jax.ShapeDtypeStruct((N, D), F32)] * 2,
    )(z2, emb, win)


@functools.partial(
    pl.kernel,
    out_type=jax.ShapeDtypeStruct((NC, N, D), F32),
    mesh=plsc.VectorSubcoreMesh(core_axis_name="c", subcore_axis_name="s",
                                num_cores=NC, num_subcores=NS),
    compiler_params=pltpu.CompilerParams(
        needs_layout_passes=False, use_tc_tiling_on_sc=False),
    scratch_types=[
        pltpu.VMEM((CAP,), jnp.int32),
        pltpu.VMEM((NCHMAX, CHUNK), jnp.int32),
        pltpu.VMEM((NCHMAX, CHUNK), jnp.int32),
        pltpu.VMEM((16,), jnp.int32),
        pltpu.VMEM((NRP * 16,), jnp.int32),
        pltpu.VMEM((ROWS_PER_TILE,), jnp.int32),
        pltpu.VMEM((ROWS_PER_TILE, D), F32),
        pltpu.VMEM((CHUNK, D), F32),
        pltpu.VMEM((CHUNK, D), F32),
        pltpu.VMEM((CHUNK, D), F32),
        pltpu.VMEM((CHUNK, D), F32),
        pltpu.VMEM((CHUNK, D), F32),
        pltpu.VMEM_SHARED((N, D), F32),
        pltpu.SemaphoreType.DMA,
        pltpu.SemaphoreType.DMA,
        pltpu.SemaphoreType.DMA,
        pltpu.SemaphoreType.DMA,
        pltpu.SemaphoreType.DMA,
        pltpu.SemaphoreType.DMA,
    ],
)
def _scatter(h, w3, eidx, cnts, rpt, agg_o,
             eflat_v, iidx_v, jidx_v, cnt_v, rpt_v, own_v, hown_v,
             hj0_v, hj1_v, m20_v, m21_v, w_v, agg_s,
             gj0, gj1, s0i, s0j, s1i, s1j):
    cid = lax.axis_index("c")
    sid = lax.axis_index("s")
    wid = sid * NC + cid
    hjs = (hj0_v, hj1_v)
    m2s = (m20_v, m21_v)
    gsems = (gj0, gj1)
    ssems = ((s0i, s0j), (s1i, s1j))

    pltpu.sync_copy(eidx.at[wid], eflat_v)
    pltpu.sync_copy(cnts.at[wid], cnt_v)
    pltpu.sync_copy(rpt.at[wid], rpt_v)

    for t in range(ROWS_PER_TILE // 16):
        own_v[pl.ds(t * 16, 16)] = (
            wid + (t * 16 + lax.iota(jnp.int32, 16)) * NW)
    pltpu.sync_copy(h.at[own_v], hown_v)

    def split_body(c, _):
        for t in range(CHUNK // 16):
            v = eflat_v[pl.ds(c * CHUNK + t * 16, 16)]
            iidx_v[c, pl.ds(t * 16, 16)] = v >> 11
            jidx_v[c, pl.ds(t * 16, 16)] = v & (N - 1)
        return 0

    lax.fori_loop(0, NCHMAX, split_body, 0)

    def zero_body(r, _):
        for t in range(D // 16):
            hj0_v[r, pl.ds(t * 16, 16)] = jnp.zeros((16,), F32)
        return 0

    lax.fori_loop(0, CHUNK, zero_body, 0)
    pltpu.sync_copy(hj0_v, agg_s.at[pl.ds(sid * CHUNK, CHUNK)])
    plsc.subcore_barrier()

    nch = jnp.max((cnt_v[...] + (CHUNK - 1)) >> 7)

    def start_gather(c, s):
        pltpu.make_async_copy(h.at[jidx_v.at[c]], hjs[s], gsems[s]).start()

    def wait_gather(c, s):
        pltpu.make_async_copy(h.at[jidx_v.at[c]], hjs[s], gsems[s]).wait()

    def start_scatters(c, s):
        pltpu.async_copy(hjs[s], agg_s.at[iidx_v.at[c]], ssems[s][0],
                         add=True)
        pltpu.async_copy(m2s[s], agg_s.at[jidx_v.at[c]], ssems[s][1],
                         add=True)

    def wait_scatters(c, s):
        pltpu.make_async_copy(hjs[s], agg_s.at[iidx_v.at[c]],
                              ssems[s][0]).wait()
        pltpu.make_async_copy(m2s[s], agg_s.at[jidx_v.at[c]],
                              ssems[s][1]).wait()

    @pl.when(nch > 0)
    def _():
        start_gather(0, 0)

    def process(c, s):
        wait_gather(c, s)
        pltpu.sync_copy(w3.at[wid, pl.ds(c * CHUNK, CHUNK)], w_v)
        hj_v = hjs[s]
        m2_v = m2s[s]

        base = c * CHUNK

        def irow_body(rr, _):
            lo = jnp.max(rpt_v[pl.ds(rr * 16, 16)])
            hi = jnp.max(rpt_v[pl.ds((rr + 1) * 16, 16)])
            lo = jnp.maximum(lo, base)
            hi = jnp.minimum(hi, base + CHUNK)

            @pl.when(hi > lo)
            def _():
                def seg_body(k, _):
                    r = k - base
                    for t in range(D // 16):
                        sl = pl.ds(t * 16, 16)
                        m2_v[r, sl] = hown_v[rr, sl] * w_v[r, sl]
                    return 0

                lax.fori_loop(lo, hi, seg_body, 0)

            return 0

        lax.fori_loop(0, ROWS_PER_TILE, irow_body, 0)

        cnt_end = jnp.max(cnt_v[...])

        def pad_body(k, _):
            r = k - base
            for t in range(D // 16):
                m2_v[r, pl.ds(t * 16, 16)] = jnp.zeros((16,), F32)
            return 0

        lax.fori_loop(jnp.maximum(cnt_end, base), base + CHUNK, pad_body, 0)

        def mul_body(r, _):
            for t in range(D // 16):
                sl = pl.ds(t * 16, 16)
                hj_v[r, sl] = hj_v[r, sl] * w_v[r, sl]
            return 0

        lax.fori_loop(0, CHUNK, mul_body, 0)
        start_scatters(c, s)

        @pl.when(c + 1 < nch)
        def _():
            @pl.when(c >= 1)
            def _():
                wait_scatters(c - 1, 1 - s)

            start_gather(c + 1, 1 - s)

    def chunk_body(c, _):
        @pl.when((c & 1) == 0)
        def _():
            process(c, 0)

        @pl.when((c & 1) == 1)
        def _():
            process(c, 1)

        return 0

    lax.fori_loop(0, nch, chunk_body, 0)

    nch_even = (nch & 1) == 0

    @pl.when((nch >= 2) & nch_even)
    def _():
        wait_scatters(nch - 2, 0)

    @pl.when((nch >= 2) & ~nch_even)
    def _():
        wait_scatters(nch - 2, 1)

    @pl.when((nch >= 1) & ~nch_even)
    def _():
        wait_scatters(nch - 1, 0)

    @pl.when((nch >= 1) & nch_even)
    def _():
        wait_scatters(nch - 1, 1)

    plsc.subcore_barrier()
    pltpu.sync_copy(agg_s.at[pl.ds(sid * CHUNK, CHUNK)],
                    agg_o.at[cid, pl.ds(sid * CHUNK, CHUNK)])


def _node_body(aggs_ref, x_ref, wd1_ref, bd1_ref, wd2_ref, bd2_ref,
               win_ref, xo_ref, ho_ref):
    agg = aggs_ref[0] + aggs_ref[1]
    t = _ssp(jnp.dot(agg, wd1_ref[...], preferred_element_type=F32)
             + bd1_ref[...])
    x = x_ref[...] + jnp.dot(t, wd2_ref[...], preferred_element_type=F32) \
        + bd2_ref[...]
    xo_ref[...] = x
    ho_ref[...] = jnp.dot(x, win_ref[...], preferred_element_type=F32)


def _node(aggs, x, wd1, bd1, wd2, bd2, win):
    return pl.pallas_call(
        _node_body,
        out_shape=[jax.ShapeDtypeStruct((N, D), F32)] * 2,
    )(aggs, x, wd1, bd1, wd2, bd2, win)


def _final_body(aggs_ref, x_ref, wd1_ref, bd1_ref, wd2_ref, bd2_ref,
                wr1_ref, br1_ref, wr2_ref, br2_ref, out_ref):
    agg = aggs_ref[0] + aggs_ref[1]
    t = _ssp(jnp.dot(agg, wd1_ref[...], preferred_element_type=F32)
             + bd1_ref[...])
    x = x_ref[...] + jnp.dot(t, wd2_ref[...], preferred_element_type=F32) \
        + bd2_ref[...]
    ea = jnp.dot(_ssp(jnp.dot(x, wr1_ref[...], preferred_element_type=F32)
                      + br1_ref[...]),
                 wr2_ref[...], preferred_element_type=F32) + br2_ref[...]
    out_ref[...] = jnp.sum(ea).reshape(1, 1)


def _final(aggs, x, wd1, bd1, wd2, bd2, wr1, br1, wr2, br2):
    return pl.pallas_call(
        _final_body,
        out_shape=jax.ShapeDtypeStruct((1, 1), F32),
    )(aggs, x, wd1, bd1, wd2, bd2, wr1, br1, wr2, br2)


def kernel(xyz, params, z):
    xyz = xyz.astype(F32)
    dm = _distmat(xyz, xyz.T)
    eidx, edist, cnts, rpt = _compact(dm)

    wf1s = jnp.stack([params['Wf1_%d' % l] for l in range(NL)])
    bf1s = jnp.stack([params['bf1_%d' % l].reshape(1, D) for l in range(NL)])
    wf2s = jnp.stack([params['Wf2_%d' % l] for l in range(NL)])
    bf2s = jnp.stack([params['bf2_%d' % l].reshape(1, D) for l in range(NL)])
    ws = _edge_mlp(edist.reshape(E, 1), wf1s, bf1s, wf2s, bf2s)

    x, h = _embed_h(z.astype(jnp.int32).reshape(N, 1), params['embed'],
                    params['Win_0'])
    out = None
    for l in range(NL):
        w3 = ws[l].reshape(NW, CAP, D)
        aggs = _scatter(h, w3, eidx, cnts, rpt)
        wd1 = params['Wd1_%d' % l]
        bd1 = params['bd1_%d' % l].reshape(1, D)
        wd2 = params['Wd2_%d' % l]
        bd2 = params['bd2_%d' % l].reshape(1, D)
        if l < NL - 1:
            x, h = _node(aggs, x, wd1, bd1, wd2, bd2,
                         params['Win_%d' % (l + 1)])
        else:
            out = _final(aggs, x, wd1, bd1, wd2, bd2,
                         params['Wr1'], params['br1'].reshape(1, 64),
                         params['Wr2'], params['br2'].reshape(1, 1))
    return out[0, 0]


# speedup vs baseline: 1.2294x; 1.0153x over previous
import functools

import jax
import jax.numpy as jnp
import numpy as np
from jax import lax
from jax.experimental import pallas as pl
from jax.experimental.pallas import tpu as pltpu
from jax.experimental.pallas import tpu_sc as plsc

N = 2048
D = 128
NG = 64
NL = 3
NT = 100
CUT = 5.0
CELL = 30.0

NC = 2
NS = 16
NW = NC * NS
CAP = 2048
E = NW * CAP
CHUNK = 128
NCHMAX = CAP // CHUNK
ROWS_PER_TILE = N // NW

LOG2 = float(np.log(2.0))
F32 = jnp.float32


def _ssp(x):
    return jnp.maximum(x, 0.0) + jnp.log(1.0 + jnp.exp(-jnp.abs(x))) - LOG2


_BR = 256


def _dist_body(xi_ref, xjt_ref, out_ref):
    p = pl.program_id(0)
    xi = xi_ref[...]
    xjt = xjt_ref[...]
    dsq = jnp.zeros((_BR, N), F32)
    for k in range(3):
        dk = xjt[k:k + 1, :] - xi[:, k:k + 1]
        off = -(dk >= 0.5 * CELL).astype(F32) + (dk < -0.5 * CELL).astype(F32)
        dk = dk + off * CELL
        dsq = dsq + dk * dk
    rows = p * _BR + lax.broadcasted_iota(jnp.int32, (_BR, N), 0)
    cols = lax.broadcasted_iota(jnp.int32, (_BR, N), 1)
    mask = (cols > rows) & (dsq < CUT * CUT) & (dsq != 0.0)
    out_ref[...] = jnp.where(mask, jnp.sqrt(dsq + 1e-12), -1.0)


def _distmat(xyz, xyzt):
    return pl.pallas_call(
        _dist_body,
        grid=(N // _BR,),
        in_specs=[
            pl.BlockSpec((_BR, 3), lambda p: (p, 0)),
            pl.BlockSpec((3, N), lambda p: (0, 0)),
        ],
        out_specs=pl.BlockSpec((_BR, N), lambda p: (p, 0)),
        out_shape=jax.ShapeDtypeStruct((N, N), F32),
    )(xyz, xyzt)


NRP = ROWS_PER_TILE + 1


@functools.partial(
    pl.kernel,
    out_type=[
        jax.ShapeDtypeStruct((NW, CAP), jnp.int32),
        jax.ShapeDtypeStruct((NW, CAP), F32),
        jax.ShapeDtypeStruct((NW, 16), jnp.int32),
        jax.ShapeDtypeStruct((NW, NRP * 16), jnp.int32),
    ],
    mesh=plsc.VectorSubcoreMesh(core_axis_name="c", subcore_axis_name="s",
                                num_cores=NC, num_subcores=NS),
    compiler_params=pltpu.CompilerParams(
        needs_layout_passes=False, use_tc_tiling_on_sc=False),
    scratch_types=[
        pltpu.VMEM((N,), F32),
        pltpu.VMEM((CAP + 16,), jnp.int32),
        pltpu.VMEM((CAP + 16,), F32),
        pltpu.VMEM((16,), jnp.int32),
        pltpu.VMEM((NRP * 16,), jnp.int32),
    ],
)
def _compact(dm, eidx_o, edist_o, cnt_o, rpt_o,
             row_v, eidx_v, edist_v, cnt_v, rpt_v):
    cid = lax.axis_index("c")
    sid = lax.axis_index("s")
    wid = sid * NC + cid

    def init_body(k, _):
        eidx_v[pl.ds(k * 16, 16)] = jnp.zeros((16,), jnp.int32)
        edist_v[pl.ds(k * 16, 16)] = jnp.full((16,), -1.0, F32)
        return 0

    lax.fori_loop(0, (CAP + 16) // 16, init_body, 0)
    rpt_v[pl.ds(0, 16)] = jnp.zeros((16,), jnp.int32)

    def row_body(rr, p):
        r = wid + rr * NW
        pltpu.sync_copy(dm.at[r], row_v)
        cstart = (r + 1) >> 4

        def chunk_body(c, p):
            v = row_v[pl.ds(c * 16, 16)]
            m = v >= 0.0
            cnt = jnp.sum(m.astype(jnp.int32))
            idx = r * N + c * 16 + lax.iota(jnp.int32, 16)
            pc = jnp.minimum(p, CAP)

            @pl.when(cnt > 0)
            def _():
                plsc.store_compressed(eidx_v.at[pl.ds(pc, 16)], idx, mask=m)
                plsc.store_compressed(edist_v.at[pl.ds(pc, 16)], v, mask=m)

            return p + cnt

        p = lax.fori_loop(cstart, N // 16, chunk_body, p)
        rpt_v[pl.ds((rr + 1) * 16, 16)] = (
            jnp.zeros((16,), jnp.int32) + jnp.minimum(p, CAP))
        return p

    p = lax.fori_loop(0, ROWS_PER_TILE, row_body, jnp.int32(0))
    cnt_v[...] = jnp.zeros((16,), jnp.int32) + jnp.minimum(p, CAP)
    pltpu.sync_copy(eidx_v.at[pl.ds(0, CAP)], eidx_o.at[wid])
    pltpu.sync_copy(edist_v.at[pl.ds(0, CAP)], edist_o.at[wid])
    pltpu.sync_copy(cnt_v, cnt_o.at[wid])
    pltpu.sync_copy(rpt_v, rpt_o.at[wid])


_BE = 4096


def _edge_body(d_ref, wf1_ref, bf1_ref, wf2_ref, bf2_ref, o_ref):
    d = d_ref[...]
    width = CUT / (NG - 1)
    cent = lax.broadcasted_iota(jnp.int32, (1, NG), 1).astype(F32) * width
    e = jnp.exp(-0.5 * ((d - cent) / width) ** 2)
    fcut = 0.5 * (jnp.cos(np.pi / CUT * d) + 1.0)
    valid = (d >= 0.0).astype(F32)
    scale = fcut * valid
    t = _ssp(jnp.dot(e, wf1_ref[...], preferred_element_type=F32)
             + bf1_ref[...])
    t = jnp.dot(t, wf2_ref[...], preferred_element_type=F32) + bf2_ref[...]
    o_ref[...] = t * scale


def _edge_mlp(edist, wf1, bf1, wf2, bf2):
    return pl.pallas_call(
        _edge_body,
        grid=(E // _BE,),
        in_specs=[
            pl.BlockSpec((_BE, 1), lambda p: (p, 0)),
            pl.BlockSpec((NG, D), lambda p: (0, 0)),
            pl.BlockSpec((1, D), lambda p: (0, 0)),
            pl.BlockSpec((D, D), lambda p: (0, 0)),
            pl.BlockSpec((1, D), lambda p: (0, 0)),
        ],
        out_specs=pl.BlockSpec((_BE, D), lambda p: (p, 0)),
        out_shape=jax.ShapeDtypeStruct((E, D), F32),
    )(edist, wf1, bf1, wf2, bf2)


def _embed_body(z_ref, emb_ref, win_ref, x_ref, h_ref):
    z = z_ref[...]
    oh = (z == lax.broadcasted_iota(jnp.int32, (1, NT), 1)).astype(F32)
    x = jnp.dot(oh, emb_ref[...], preferred_element_type=F32)
    x_ref[...] = x
    h_ref[...] = jnp.dot(x, win_ref[...], preferred_element_type=F32)


def _embed_h(z2, emb, win):
    return pl.pallas_call(
        _embed_body,
        out_shape=[jax.ShapeDtypeStruct((N, D), F32)] * 2,
    )(z2, emb, win)


@functools.partial(
    pl.kernel,
    out_type=jax.ShapeDtypeStruct((NC, N, D), F32),
    mesh=plsc.VectorSubcoreMesh(core_axis_name="c", subcore_axis_name="s",
                                num_cores=NC, num_subcores=NS),
    compiler_params=pltpu.CompilerParams(
        needs_layout_passes=False, use_tc_tiling_on_sc=False),
    scratch_types=[
        pltpu.VMEM((CAP,), jnp.int32),
        pltpu.VMEM((NCHMAX, CHUNK), jnp.int32),
        pltpu.VMEM((NCHMAX, CHUNK), jnp.int32),
        pltpu.VMEM((16,), jnp.int32),
        pltpu.VMEM((NRP * 16,), jnp.int32),
        pltpu.VMEM((ROWS_PER_TILE,), jnp.int32),
        pltpu.VMEM((ROWS_PER_TILE, D), F32),
        pltpu.VMEM((CHUNK, D), F32),
        pltpu.VMEM((CHUNK, D), F32),
        pltpu.VMEM((CHUNK, D), F32),
        pltpu.VMEM((CHUNK, D), F32),
        pltpu.VMEM((CHUNK, D), F32),
        pltpu.VMEM_SHARED((N, D), F32),
        pltpu.SemaphoreType.DMA,
        pltpu.SemaphoreType.DMA,
        pltpu.SemaphoreType.DMA,
        pltpu.SemaphoreType.DMA,
        pltpu.SemaphoreType.DMA,
        pltpu.SemaphoreType.DMA,
    ],
)
def _scatter(h, w3, eidx, cnts, rpt, agg_o,
             eflat_v, iidx_v, jidx_v, cnt_v, rpt_v, own_v, hown_v,
             hj0_v, hj1_v, m20_v, m21_v, w_v, agg_s,
             gj0, gj1, s0i, s0j, s1i, s1j):
    cid = lax.axis_index("c")
    sid = lax.axis_index("s")
    wid = sid * NC + cid
    hjs = (hj0_v, hj1_v)
    m2s = (m20_v, m21_v)
    gsems = (gj0, gj1)
    ssems = ((s0i, s0j), (s1i, s1j))

    pltpu.sync_copy(eidx.at[wid], eflat_v)
    pltpu.sync_copy(cnts.at[wid], cnt_v)
    pltpu.sync_copy(rpt.at[wid], rpt_v)

    for t in range(ROWS_PER_TILE // 16):
        own_v[pl.ds(t * 16, 16)] = (
            wid + (t * 16 + lax.iota(jnp.int32, 16)) * NW)
    pltpu.sync_copy(h.at[own_v], hown_v)

    def split_body(c, _):
        for t in range(CHUNK // 16):
            v = eflat_v[pl.ds(c * CHUNK + t * 16, 16)]
            iidx_v[c, pl.ds(t * 16, 16)] = v >> 11
            jidx_v[c, pl.ds(t * 16, 16)] = v & (N - 1)
        return 0

    lax.fori_loop(0, NCHMAX, split_body, 0)

    def zero_body(r, _):
        for t in range(D // 16):
            hj0_v[r, pl.ds(t * 16, 16)] = jnp.zeros((16,), F32)
        return 0

    lax.fori_loop(0, CHUNK, zero_body, 0)
    pltpu.sync_copy(hj0_v, agg_s.at[pl.ds(sid * CHUNK, CHUNK)])
    plsc.subcore_barrier()

    nch = jnp.max((cnt_v[...] + (CHUNK - 1)) >> 7)

    def start_gather(c, s):
        pltpu.make_async_copy(h.at[jidx_v.at[c]], hjs[s], gsems[s]).start()

    def wait_gather(c, s):
        pltpu.make_async_copy(h.at[jidx_v.at[c]], hjs[s], gsems[s]).wait()

    def start_scatters(c, s):
        pltpu.async_copy(hjs[s], agg_s.at[iidx_v.at[c]], ssems[s][0],
                         add=True)
        pltpu.async_copy(m2s[s], agg_s.at[jidx_v.at[c]], ssems[s][1],
                         add=True)

    def wait_scatters(c, s):
        pltpu.make_async_copy(hjs[s], agg_s.at[iidx_v.at[c]],
                              ssems[s][0]).wait()
        pltpu.make_async_copy(m2s[s], agg_s.at[jidx_v.at[c]],
                              ssems[s][1]).wait()

    @pl.when(nch > 0)
    def _():
        start_gather(0, 0)

    def process(c, s):
        wait_gather(c, s)
        pltpu.sync_copy(w3.at[wid, pl.ds(c * CHUNK, CHUNK)], w_v)
        hj_v = hjs[s]
        m2_v = m2s[s]

        base = c * CHUNK

        def irow_body(rr, _):
            lo = jnp.max(rpt_v[pl.ds(rr * 16, 16)])
            hi = jnp.max(rpt_v[pl.ds((rr + 1) * 16, 16)])
            lo = jnp.maximum(lo, base)
            hi = jnp.minimum(hi, base + CHUNK)

            @pl.when(hi > lo)
            def _():
                def seg_body(k, _):
                    r = k - base
                    for t in range(D // 16):
                        sl = pl.ds(t * 16, 16)
                        m2_v[r, sl] = hown_v[rr, sl] * w_v[r, sl]
                    return 0

                lax.fori_loop(lo, hi, seg_body, 0)

            return 0

        lax.fori_loop(0, ROWS_PER_TILE, irow_body, 0)

        cnt_end = jnp.max(cnt_v[...])

        def pad_body(k, _):
            r = k - base
            for t in range(D // 16):
                m2_v[r, pl.ds(t * 16, 16)] = jnp.zeros((16,), F32)
            return 0

        lax.fori_loop(jnp.maximum(cnt_end, base), base + CHUNK, pad_body, 0)

        def mul_body(r, _):
            for t in range(D // 16):
                sl = pl.ds(t * 16, 16)
                hj_v[r, sl] = hj_v[r, sl] * w_v[r, sl]
            return 0

        lax.fori_loop(0, CHUNK, mul_body, 0)
        start_scatters(c, s)

        @pl.when(c + 1 < nch)
        def _():
            @pl.when(c >= 1)
            def _():
                wait_scatters(c - 1, 1 - s)

            start_gather(c + 1, 1 - s)

    def chunk_body(c, _):
        @pl.when((c & 1) == 0)
        def _():
            process(c, 0)

        @pl.when((c & 1) == 1)
        def _():
            process(c, 1)

        return 0

    lax.fori_loop(0, nch, chunk_body, 0)

    nch_even = (nch & 1) == 0

    @pl.when((nch >= 2) & nch_even)
    def _():
        wait_scatters(nch - 2, 0)

    @pl.when((nch >= 2) & ~nch_even)
    def _():
        wait_scatters(nch - 2, 1)

    @pl.when((nch >= 1) & ~nch_even)
    def _():
        wait_scatters(nch - 1, 0)

    @pl.when((nch >= 1) & nch_even)
    def _():
        wait_scatters(nch - 1, 1)

    plsc.subcore_barrier()
    pltpu.sync_copy(agg_s.at[pl.ds(sid * CHUNK, CHUNK)],
                    agg_o.at[cid, pl.ds(sid * CHUNK, CHUNK)])


def _node_body(aggs_ref, x_ref, wd1_ref, bd1_ref, wd2_ref, bd2_ref,
               win_ref, xo_ref, ho_ref):
    agg = aggs_ref[0] + aggs_ref[1]
    t = _ssp(jnp.dot(agg, wd1_ref[...], preferred_element_type=F32)
             + bd1_ref[...])
    x = x_ref[...] + jnp.dot(t, wd2_ref[...], preferred_element_type=F32) \
        + bd2_ref[...]
    xo_ref[...] = x
    ho_ref[...] = jnp.dot(x, win_ref[...], preferred_element_type=F32)


def _node(aggs, x, wd1, bd1, wd2, bd2, win):
    return pl.pallas_call(
        _node_body,
        out_shape=[jax.ShapeDtypeStruct((N, D), F32)] * 2,
    )(aggs, x, wd1, bd1, wd2, bd2, win)


def _final_body(aggs_ref, x_ref, wd1_ref, bd1_ref, wd2_ref, bd2_ref,
                wr1_ref, br1_ref, wr2_ref, br2_ref, out_ref):
    agg = aggs_ref[0] + aggs_ref[1]
    t = _ssp(jnp.dot(agg, wd1_ref[...], preferred_element_type=F32)
             + bd1_ref[...])
    x = x_ref[...] + jnp.dot(t, wd2_ref[...], preferred_element_type=F32) \
        + bd2_ref[...]
    ea = jnp.dot(_ssp(jnp.dot(x, wr1_ref[...], preferred_element_type=F32)
                      + br1_ref[...]),
                 wr2_ref[...], preferred_element_type=F32) + br2_ref[...]
    out_ref[...] = jnp.sum(ea).reshape(1, 1)


def _final(aggs, x, wd1, bd1, wd2, bd2, wr1, br1, wr2, br2):
    return pl.pallas_call(
        _final_body,
        out_shape=jax.ShapeDtypeStruct((1, 1), F32),
    )(aggs, x, wd1, bd1, wd2, bd2, wr1, br1, wr2, br2)


def kernel(xyz, params, z):
    xyz = xyz.astype(F32)
    dm = _distmat(xyz, xyz.T)
    eidx, edist, cnts, rpt = _compact(dm)

    ed = edist.reshape(E, 1)
    x, h = _embed_h(z.astype(jnp.int32).reshape(N, 1), params['embed'],
                    params['Win_0'])
    out = None
    for l in range(NL):
        wl = _edge_mlp(ed, params['Wf1_%d' % l],
                       params['bf1_%d' % l].reshape(1, D),
                       params['Wf2_%d' % l],
                       params['bf2_%d' % l].reshape(1, D))
        w3 = wl.reshape(NW, CAP, D)
        aggs = _scatter(h, w3, eidx, cnts, rpt)
        wd1 = params['Wd1_%d' % l]
        bd1 = params['bd1_%d' % l].reshape(1, D)
        wd2 = params['Wd2_%d' % l]
        bd2 = params['bd2_%d' % l].reshape(1, D)
        if l < NL - 1:
            x, h = _node(aggs, x, wd1, bd1, wd2, bd2,
                         params['Win_%d' % (l + 1)])
        else:
            out = _final(aggs, x, wd1, bd1, wd2, bd2,
                         params['Wr1'], params['br1'].reshape(1, 64),
                         params['Wr2'], params['br2'].reshape(1, 1))
    return out[0, 0]
